# baseline (device time: 47632 ns/iter reference)
import jax
import jax.numpy as jnp
from jax import lax
from jax.experimental import pallas as pl
from jax.experimental.pallas import tpu as pltpu

N_DEV = 4
B = 2
SQ = 256
SKV = 256
HQ = 16
DH = 64
DM = 512
HPS = HQ // N_DEV
WQC = HPS * DH
BLK = 64


def kernel(x, Wq, K_ext, V_ext, Wo):
    def body(x_ref, wq_ref, k_ref, v_ref, wo_ref, out_ref,
             wq_full, wo_full, wq_send, wq_recv, wo_send, wo_recv):
        my = lax.axis_index("i")
        right = lax.rem(my + 1, N_DEV)
        left = lax.rem(my + 3, N_DEV)
        opp = lax.rem(my + 2, N_DEV)
        peers = [right, left, opp]

        barrier = pltpu.get_barrier_semaphore()
        for nbr in peers:
            pl.semaphore_signal(
                barrier, inc=1,
                device_id=(nbr,), device_id_type=pl.DeviceIdType.MESH,
            )
        pl.semaphore_wait(barrier, len(peers))

        wq_full[pl.ds(my * DM, DM), :] = wq_ref[:, :]
        wo_full[pl.ds(my * WQC, WQC), :] = wo_ref[:, :]

        rdmas = []
        for idx, tgt in enumerate(peers):
            r_wq = pltpu.make_async_remote_copy(
                src_ref=wq_ref,
                dst_ref=wq_full.at[pl.ds(my * DM, DM), :],
                send_sem=wq_send.at[idx],
                recv_sem=wq_recv.at[idx],
                device_id=(tgt,),
                device_id_type=pl.DeviceIdType.MESH,
            )
            r_wo = pltpu.make_async_remote_copy(
                src_ref=wo_ref,
                dst_ref=wo_full.at[pl.ds(my * WQC, WQC), :],
                send_sem=wo_send.at[idx],
                recv_sem=wo_recv.at[idx],
                device_id=(tgt,),
                device_id_type=pl.DeviceIdType.MESH,
            )
            r_wq.start()
            r_wo.start()
            rdmas.append((r_wq, r_wo))

        r_iota = lax.broadcasted_iota(jnp.int32, (SQ, SKV), 0)
        c_iota = lax.broadcasted_iota(jnp.int32, (SQ, SKV), 1)
        qb = my * (SQ // BLK) + r_iota // BLK
        kb = c_iota // BLK
        mask = (qb == kb) | (kb == 0) | (lax.rem(qb + kb, 3) == 0)

        for r_wq, r_wo in rdmas:
            r_wq.wait()
            r_wo.wait()

        for b in range(B):
            x_b = x_ref[b]
            k_b = k_ref[b]
            v_b = v_ref[b]
            acc = jnp.zeros((SQ, DM), jnp.float32)
            for j in range(N_DEV):
                wq_j = wq_full[DM * j:DM * (j + 1), :]
                q_j = jnp.dot(x_b, wq_j, preferred_element_type=jnp.float32)
                ctx_cols = []
                for hh in range(HPS):
                    head = HPS * j + hh
                    qh = q_j[:, DH * hh:DH * (hh + 1)]
                    kh = k_b[:, DH * head:DH * (head + 1)]
                    vh = v_b[:, DH * head:DH * (head + 1)]
                    s = lax.dot_general(
                        qh, kh, (((1,), (1,)), ((), ())),
                        preferred_element_type=jnp.float32,
                    )
                    s = jnp.where(mask, s * 0.125, -1e9)
                    m = jnp.max(s, axis=-1, keepdims=True)
                    w = jnp.exp(s - m)
                    w = w / jnp.sum(w, axis=-1, keepdims=True)
                    ctx_cols.append(
                        jnp.dot(w, vh, preferred_element_type=jnp.float32)
                    )
                ctx_j = jnp.concatenate(ctx_cols, axis=1)
                acc = acc + jnp.dot(
                    ctx_j, wo_full[WQC * j:WQC * (j + 1), :],
                    preferred_element_type=jnp.float32,
                )
            out_ref[b] = acc

    k2 = K_ext.reshape(B, SKV, HQ * DH)
    v2 = V_ext.reshape(B, SKV, HQ * DH)

    return pl.pallas_call(
        body,
        out_shape=jax.ShapeDtypeStruct((B, SQ, DM), jnp.float32),
        in_specs=[pl.BlockSpec(memory_space=pltpu.VMEM)] * 5,
        out_specs=pl.BlockSpec(memory_space=pltpu.VMEM),
        scratch_shapes=[
            pltpu.VMEM((N_DEV * DM, WQC), jnp.float32),
            pltpu.VMEM((N_DEV * WQC, DM), jnp.float32),
            pltpu.SemaphoreType.DMA((3,)),
            pltpu.SemaphoreType.DMA((3,)),
            pltpu.SemaphoreType.DMA((3,)),
            pltpu.SemaphoreType.DMA((3,)),
        ],
        compiler_params=pltpu.CompilerParams(collective_id=0),
    )(x, Wq, k2, v2, Wo)


# device time: 27227 ns/iter; 1.7494x vs baseline; 1.7494x over previous
import jax
import jax.numpy as jnp
from jax import lax
from jax.experimental import pallas as pl
from jax.experimental.pallas import tpu as pltpu

N_DEV = 4
B = 2
SQ = 256
SKV = 256
HQ = 16
DH = 64
DM = 512
HPS = HQ // N_DEV
WQC = HPS * DH
BLK = 64


def kernel(x, Wq, K_ext, V_ext, Wo):
    def body(x_ref, wq_ref, k_ref, v_ref, wo_ref, out_ref,
             wq_slots, wo_slots, wq_send, wq_recv, wo_send, wo_recv):
        my = lax.axis_index("i")
        right = lax.rem(my + 1, N_DEV)
        left = lax.rem(my + 3, N_DEV)
        opp = lax.rem(my + 2, N_DEV)

        barrier = pltpu.get_barrier_semaphore()
        for nbr in (right, left, opp):
            pl.semaphore_signal(
                barrier, inc=1,
                device_id=(nbr,), device_id_type=pl.DeviceIdType.MESH,
            )
        pl.semaphore_wait(barrier, 3)

        wq_slots[0] = wq_ref[:, :].astype(jnp.bfloat16)
        wo_slots[0] = wo_ref[:, :].astype(jnp.bfloat16)

        rdmas = {}
        for s, tgt in ((1, left), (3, right), (2, opp)):
            r_wq = pltpu.make_async_remote_copy(
                src_ref=wq_slots.at[0],
                dst_ref=wq_slots.at[s],
                send_sem=wq_send.at[s],
                recv_sem=wq_recv.at[s],
                device_id=(tgt,),
                device_id_type=pl.DeviceIdType.MESH,
            )
            r_wo = pltpu.make_async_remote_copy(
                src_ref=wo_slots.at[0],
                dst_ref=wo_slots.at[s],
                send_sem=wo_send.at[s],
                recv_sem=wo_recv.at[s],
                device_id=(tgt,),
                device_id_type=pl.DeviceIdType.MESH,
            )
            r_wq.start()
            r_wo.start()
            rdmas[s] = (r_wq, r_wo)


        r_iota = lax.broadcasted_iota(jnp.int32, (SQ, SKV), 0)
        c_iota = lax.broadcasted_iota(jnp.int32, (SQ, SKV), 1)
        qb = my * (SQ // BLK) + r_iota // BLK
        kb = c_iota // BLK
        mask = (qb == kb) | (kb == 0) | (lax.rem(qb + kb, 3) == 0)

        shift = lax.rem((N_DEV - my) * WQC, N_DEV * WQC)
        k_roll = [pltpu.roll(k_ref[b], shift, 1) for b in range(B)]
        v_roll = [pltpu.roll(v_ref[b], shift, 1) for b in range(B)]
        x_bf = [x_ref[b].astype(jnp.bfloat16) for b in range(B)]

        def chunk_contrib(s, b):
            wq_s = wq_slots[s]
            q_s = jnp.dot(x_bf[b], wq_s,
                          preferred_element_type=jnp.float32)
            ctx_cols = []
            for hh in range(HPS):
                qh = q_s[:, DH * hh:DH * (hh + 1)]
                col = WQC * s + DH * hh
                kh = k_roll[b][:, col:col + DH]
                vh = v_roll[b][:, col:col + DH]
                sc = lax.dot_general(
                    qh, kh, (((1,), (1,)), ((), ())),
                    preferred_element_type=jnp.float32,
                )
                sc = jnp.where(mask, sc * 0.125, -1e9)
                m = jnp.max(sc, axis=-1, keepdims=True)
                w = jnp.exp(sc - m)
                w = w / jnp.sum(w, axis=-1, keepdims=True)
                ctx_cols.append(
                    jnp.dot(w, vh, preferred_element_type=jnp.float32)
                )
            ctx = jnp.concatenate(ctx_cols, axis=1)
            return jnp.dot(ctx.astype(jnp.bfloat16), wo_slots[s],
                           preferred_element_type=jnp.float32)

        acc = [chunk_contrib(0, b) for b in range(B)]

        for s in (1, 3, 2):
            rdmas[s][0].wait_recv()
            rdmas[s][1].wait_recv()
            for b in range(B):
                acc[b] = acc[b] + chunk_contrib(s, b)

        for b in range(B):
            out_ref[b] = acc[b]

        for s in (1, 3, 2):
            rdmas[s][0].wait_send()
            rdmas[s][1].wait_send()

    k2 = K_ext.reshape(B, SKV, HQ * DH)
    v2 = V_ext.reshape(B, SKV, HQ * DH)

    return pl.pallas_call(
        body,
        out_shape=jax.ShapeDtypeStruct((B, SQ, DM), jnp.float32),
        in_specs=[pl.BlockSpec(memory_space=pltpu.VMEM)] * 5,
        out_specs=pl.BlockSpec(memory_space=pltpu.VMEM),
        scratch_shapes=[
            pltpu.VMEM((N_DEV, DM, WQC), jnp.bfloat16),
            pltpu.VMEM((N_DEV, WQC, DM), jnp.bfloat16),
            pltpu.SemaphoreType.DMA((N_DEV,)),
            pltpu.SemaphoreType.DMA((N_DEV,)),
            pltpu.SemaphoreType.DMA((N_DEV,)),
            pltpu.SemaphoreType.DMA((N_DEV,)),
        ],
        compiler_params=pltpu.CompilerParams(collective_id=0),
    )(x, Wq, k2, v2, Wo)


# device time: 22086 ns/iter; 2.1567x vs baseline; 1.2328x over previous
import jax
import jax.numpy as jnp
from jax import lax
from jax.experimental import pallas as pl
from jax.experimental.pallas import tpu as pltpu

N_DEV = 4
B = 2
SQ = 256
SKV = 256
HQ = 16
DH = 64
DM = 512
HPS = HQ // N_DEV
WQC = HPS * DH
BLK = 64

BF = jnp.bfloat16


def kernel(x, Wq, K_ext, V_ext, Wo):
    def body(x_ref, wq_ref, k_ref, v_ref, wo_ref, out_ref,
             wq_slots, wo_slots, wq_send, wq_recv, wo_send, wo_recv):
        my = lax.axis_index("i")
        right = lax.rem(my + 1, N_DEV)
        left = lax.rem(my + 3, N_DEV)
        opp = lax.rem(my + 2, N_DEV)

        barrier = pltpu.get_barrier_semaphore()
        for nbr in (right, left, opp):
            pl.semaphore_signal(
                barrier, inc=1,
                device_id=(nbr,), device_id_type=pl.DeviceIdType.MESH,
            )
        pl.semaphore_wait(barrier, 3)

        wq_slots[0] = wq_ref[:, :].astype(BF)
        wo_slots[0] = wo_ref[:, :].astype(BF)

        def push(slots, send, recv, s, tgt):
            r = pltpu.make_async_remote_copy(
                src_ref=slots.at[0],
                dst_ref=slots.at[s],
                send_sem=send.at[s],
                recv_sem=recv.at[s],
                device_id=(tgt,),
                device_id_type=pl.DeviceIdType.MESH,
            )
            r.start()
            return r

        order = ((1, left), (3, right), (2, opp))
        r_wq = {s: push(wq_slots, wq_send, wq_recv, s, t) for s, t in order}
        r_wo = {s: push(wo_slots, wo_send, wo_recv, s, t) for s, t in order}


        r_iota = lax.broadcasted_iota(jnp.int32, (SQ, SKV), 0)
        c_iota = lax.broadcasted_iota(jnp.int32, (SQ, SKV), 1)
        qb = my * (SQ // BLK) + r_iota // BLK
        kb = c_iota // BLK
        mask = (qb == kb) | (kb == 0) | (lax.rem(qb + kb, 3) == 0)

        shift = lax.rem((N_DEV - my) * WQC, N_DEV * WQC)
        k_roll = [pltpu.roll(k_ref[b].astype(BF), shift, 1) for b in range(B)]
        v_roll = [pltpu.roll(v_ref[b].astype(BF), shift, 1) for b in range(B)]
        x_bf = [x_ref[b].astype(BF) for b in range(B)]

        def chunk_ctx(s, b):
            q_s = jnp.dot(x_bf[b], wq_slots[s],
                          preferred_element_type=jnp.float32)
            q_bf = q_s.astype(BF)
            ctx_cols = []
            for hh in range(HPS):
                qh = q_bf[:, DH * hh:DH * (hh + 1)]
                col = WQC * s + DH * hh
                kh = k_roll[b][:, col:col + DH]
                vh = v_roll[b][:, col:col + DH]
                sc = lax.dot_general(
                    qh, kh, (((1,), (1,)), ((), ())),
                    preferred_element_type=jnp.float32,
                )
                sc = jnp.where(mask, sc * 0.125, -1e9)
                m = jnp.max(sc, axis=-1, keepdims=True)
                w = jnp.exp(sc - m)
                w = (w / jnp.sum(w, axis=-1, keepdims=True)).astype(BF)
                ctx_cols.append(
                    jnp.dot(w, vh, preferred_element_type=jnp.float32)
                )
            return jnp.concatenate(ctx_cols, axis=1).astype(BF)

        def out_proj(ctx_sb, s, b):
            return jnp.dot(ctx_sb, wo_slots[s],
                           preferred_element_type=jnp.float32)

        consume = (1, 3, 2)

        ctx = {(0, b): chunk_ctx(0, b) for b in range(B)}
        acc = [out_proj(ctx[(0, b)], 0, b) for b in range(B)]

        for s in consume:
            r_wq[s].wait_recv()
            for b in range(B):
                ctx[(s, b)] = chunk_ctx(s, b)

        for s in consume:
            r_wo[s].wait_recv()
            for b in range(B):
                acc[b] = acc[b] + out_proj(ctx[(s, b)], s, b)

        for b in range(B):
            out_ref[b] = acc[b]

        for s in consume:
            r_wq[s].wait_send()
            r_wo[s].wait_send()

    k2 = K_ext.reshape(B, SKV, HQ * DH)
    v2 = V_ext.reshape(B, SKV, HQ * DH)

    return pl.pallas_call(
        body,
        out_shape=jax.ShapeDtypeStruct((B, SQ, DM), jnp.float32),
        in_specs=[pl.BlockSpec(memory_space=pltpu.VMEM)] * 5,
        out_specs=pl.BlockSpec(memory_space=pltpu.VMEM),
        scratch_shapes=[
            pltpu.VMEM((N_DEV, DM, WQC), BF),
            pltpu.VMEM((N_DEV, WQC, DM), BF),
            pltpu.SemaphoreType.DMA((N_DEV,)),
            pltpu.SemaphoreType.DMA((N_DEV,)),
            pltpu.SemaphoreType.DMA((N_DEV,)),
            pltpu.SemaphoreType.DMA((N_DEV,)),
        ],
        compiler_params=pltpu.CompilerParams(collective_id=0),
    )(x, Wq, k2, v2, Wo)
